# initial kernel scaffold (unmeasured)
import jax
import jax.numpy as jnp
from jax import lax
from jax.experimental import pallas as pl
from jax.experimental.pallas import tpu as pltpu

N_DEV = 32
LOG2 = 5
B, SQ, D, HQ, DH = 2, 128, 512, 8, 64
BSQ = B * SQ
PAY = 640


def kernel(x, Wq, Wo, K_ext, V_ext):
    skv = K_ext.shape[1]

    def body(x_ref, wq_ref, wo_ref, k_ref, v_ref, out_ref,
             acc_ref, recv_ref, send_sems, recv_sems):
        my_i = lax.axis_index("i")

        xb = x_ref[...].reshape(BSQ, D).astype(jnp.bfloat16)
        q2d = jax.lax.dot(xb, wq_ref[...].astype(jnp.bfloat16),
                          preferred_element_type=jnp.float32)
        q2d = q2d.astype(jnp.bfloat16)

        acc_ref[:, pl.ds(512, 128)] = jnp.zeros((BSQ, 128), jnp.float32)
        for b in range(B):
            for h in range(HQ):
                qbh = q2d[b * SQ:(b + 1) * SQ, h * DH:(h + 1) * DH]
                kbh = k_ref[b, :, h, :].astype(jnp.bfloat16)
                s = jax.lax.dot_general(
                    qbh, kbh, (((1,), (1,)), ((), ())),
                    preferred_element_type=jnp.float32) * 0.125
                p = jnp.exp(s)
                lsum = jnp.sum(p, axis=1, keepdims=True)
                vbh = v_ref[b, :, h, :].astype(jnp.bfloat16)
                o = jax.lax.dot(p.astype(jnp.bfloat16), vbh,
                                preferred_element_type=jnp.float32)
                acc_ref[pl.ds(b * SQ, SQ), pl.ds(h * DH, DH)] = o
                acc_ref[pl.ds(b * SQ, SQ), pl.ds(512 + h, 1)] = lsum

        for s in range(LOG2):
            partner = my_i ^ (1 << s)
            rdma = pltpu.make_async_remote_copy(
                src_ref=acc_ref,
                dst_ref=recv_ref.at[s],
                send_sem=send_sems.at[s],
                recv_sem=recv_sems.at[s],
                device_id=(partner,),
                device_id_type=pl.DeviceIdType.MESH,
            )
            rdma.start()
            rdma.wait()
            acc_ref[...] += recv_ref[s]

        accv = acc_ref[...]
        for h in range(HQ):
            acc_ref[:, pl.ds(h * DH, DH)] = (
                accv[:, h * DH:(h + 1) * DH] / accv[:, 512 + h:513 + h]
            )
        onorm = acc_ref[:, pl.ds(0, 512)].astype(jnp.bfloat16)
        res = jax.lax.dot(onorm, wo_ref[...].astype(jnp.bfloat16),
                          preferred_element_type=jnp.float32)
        out_ref[...] = res.reshape(B, SQ, D)

    return pl.pallas_call(
        body,
        out_shape=jax.ShapeDtypeStruct((B, SQ, D), jnp.float32),
        in_specs=[pl.BlockSpec(memory_space=pltpu.VMEM)] * 5,
        out_specs=pl.BlockSpec(memory_space=pltpu.VMEM),
        scratch_shapes=[
            pltpu.VMEM((BSQ, PAY), jnp.float32),
            pltpu.VMEM((LOG2, BSQ, PAY), jnp.float32),
            pltpu.SemaphoreType.DMA((LOG2,)),
            pltpu.SemaphoreType.DMA((LOG2,)),
        ],
        compiler_params=pltpu.CompilerParams(collective_id=0),
    )(x, Wq, Wo, K_ext, V_ext)


# baseline (device time: 80051 ns/iter reference)
import jax
import jax.numpy as jnp
from jax import lax
from jax.experimental import pallas as pl
from jax.experimental.pallas import tpu as pltpu

N_DEV = 32
LOG2 = 5
B, SQ, D, HQ, DH = 2, 128, 512, 8, 64
BSQ = B * SQ
PAY = 640


def kernel(x, Wq, Wo, K_ext, V_ext):
    skv = K_ext.shape[1]

    def body(x_ref, wq_ref, wo_ref, k_ref, v_ref, out_ref,
             acc_ref, recv_ref, send_sems, recv_sems):
        my_i = lax.axis_index("i")

        xb = x_ref[...].reshape(BSQ, D).astype(jnp.bfloat16)
        q2d = jax.lax.dot(xb, wq_ref[...].astype(jnp.bfloat16),
                          preferred_element_type=jnp.float32)
        q2d = q2d.astype(jnp.bfloat16)

        acc_ref[:, pl.ds(512, 128)] = jnp.zeros((BSQ, 128), jnp.float32)
        for b in range(B):
            for h in range(HQ):
                qbh = q2d[b * SQ:(b + 1) * SQ, h * DH:(h + 1) * DH]
                kbh = k_ref[b, :, h, :].astype(jnp.bfloat16)
                s = jax.lax.dot_general(
                    qbh, kbh, (((1,), (1,)), ((), ())),
                    preferred_element_type=jnp.float32) * 0.125
                p = jnp.exp(s)
                lsum = jnp.sum(p, axis=1, keepdims=True)
                vbh = v_ref[b, :, h, :].astype(jnp.bfloat16)
                o = jax.lax.dot(p.astype(jnp.bfloat16), vbh,
                                preferred_element_type=jnp.float32)
                acc_ref[pl.ds(b * SQ, SQ), pl.ds(h * DH, DH)] = o
                acc_ref[pl.ds(b * SQ, SQ), pl.ds(512 + h, 1)] = lsum

        for s in range(LOG2):
            partner = my_i ^ (1 << s)
            rdma = pltpu.make_async_remote_copy(
                src_ref=acc_ref,
                dst_ref=recv_ref.at[s],
                send_sem=send_sems.at[s],
                recv_sem=recv_sems.at[s],
                device_id=(partner,),
                device_id_type=pl.DeviceIdType.MESH,
            )
            rdma.start()
            rdma.wait()
            acc_ref[...] += recv_ref[s]

        accv = acc_ref[...]
        for h in range(HQ):
            acc_ref[:, pl.ds(h * DH, DH)] = (
                accv[:, h * DH:(h + 1) * DH] / accv[:, 512 + h:513 + h]
            )
        onorm = acc_ref[:, pl.ds(0, 512)].astype(jnp.bfloat16)
        res = jax.lax.dot(onorm, wo_ref[...].astype(jnp.bfloat16),
                          preferred_element_type=jnp.float32)
        out_ref[...] = res.reshape(B, SQ, D)

    return pl.pallas_call(
        body,
        out_shape=jax.ShapeDtypeStruct((B, SQ, D), jnp.float32),
        in_specs=[pl.BlockSpec(memory_space=pltpu.VMEM)] * 5,
        out_specs=pl.BlockSpec(memory_space=pltpu.VMEM),
        scratch_shapes=[
            pltpu.VMEM((BSQ, PAY), jnp.float32),
            pltpu.VMEM((LOG2, BSQ, PAY), jnp.float32),
            pltpu.SemaphoreType.DMA((LOG2,)),
            pltpu.SemaphoreType.DMA((LOG2,)),
        ],
    )(x, Wq, Wo, K_ext, V_ext)


# device time: 48221 ns/iter; 1.6601x vs baseline; 1.6601x over previous
import jax
import jax.numpy as jnp
from jax import lax
from jax.experimental import pallas as pl
from jax.experimental.pallas import tpu as pltpu

N_DEV = 32
LOG2 = 5
B, SQ, D, HQ, DH = 2, 128, 512, 8, 64
BSQ = B * SQ
PAY = 640


def kernel(x, Wq, Wo, K_ext, V_ext):
    skv = K_ext.shape[1]

    def body(x_ref, wq_ref, wo_ref, k_ref, v_ref, out_ref,
             acc_ref, send_ref, recv_ref, send_sems, recv_sems):
        my_i = lax.axis_index("i")

        xb = x_ref[...].reshape(BSQ, D).astype(jnp.bfloat16)
        q2d = jax.lax.dot(xb, wq_ref[...].astype(jnp.bfloat16),
                          preferred_element_type=jnp.float32)
        q2d = q2d.astype(jnp.bfloat16)

        acc_ref[:, pl.ds(512, 128)] = jnp.zeros((BSQ, 128), jnp.float32)
        for b in range(B):
            for h in range(HQ):
                qbh = q2d[b * SQ:(b + 1) * SQ, h * DH:(h + 1) * DH]
                kbh = k_ref[b, :, h, :].astype(jnp.bfloat16)
                s = jax.lax.dot_general(
                    qbh, kbh, (((1,), (1,)), ((), ())),
                    preferred_element_type=jnp.float32) * 0.125
                p = jnp.exp(s)
                lsum = jnp.sum(p, axis=1, keepdims=True)
                vbh = v_ref[b, :, h, :].astype(jnp.bfloat16)
                o = jax.lax.dot(p.astype(jnp.bfloat16), vbh,
                                preferred_element_type=jnp.float32)
                acc_ref[pl.ds(b * SQ, SQ), pl.ds(h * DH, DH)] = o
                acc_ref[pl.ds(b * SQ, SQ), pl.ds(512 + h, 1)] = lsum

        barrier = pltpu.get_barrier_semaphore()
        for s in range(LOG2):
            partner = my_i ^ (1 << s)
            pl.semaphore_signal(barrier, inc=1, device_id=(partner,),
                                device_id_type=pl.DeviceIdType.MESH)
        pl.semaphore_wait(barrier, LOG2)

        for s in range(LOG2):
            partner = my_i ^ (1 << s)
            send_ref[...] = acc_ref[...].astype(jnp.bfloat16)
            rdma = pltpu.make_async_remote_copy(
                src_ref=send_ref,
                dst_ref=recv_ref.at[s],
                send_sem=send_sems.at[s],
                recv_sem=recv_sems.at[s],
                device_id=(partner,),
                device_id_type=pl.DeviceIdType.MESH,
            )
            rdma.start()
            rdma.wait()
            acc_ref[...] += recv_ref[s].astype(jnp.float32)

        accv = acc_ref[...]
        for h in range(HQ):
            acc_ref[:, pl.ds(h * DH, DH)] = (
                accv[:, h * DH:(h + 1) * DH] / accv[:, 512 + h:513 + h]
            )
        onorm = acc_ref[:, pl.ds(0, 512)].astype(jnp.bfloat16)
        res = jax.lax.dot(onorm, wo_ref[...].astype(jnp.bfloat16),
                          preferred_element_type=jnp.float32)
        out_ref[...] = res.reshape(B, SQ, D)

    return pl.pallas_call(
        body,
        out_shape=jax.ShapeDtypeStruct((B, SQ, D), jnp.float32),
        in_specs=[pl.BlockSpec(memory_space=pltpu.VMEM)] * 5,
        out_specs=pl.BlockSpec(memory_space=pltpu.VMEM),
        scratch_shapes=[
            pltpu.VMEM((BSQ, PAY), jnp.float32),
            pltpu.VMEM((BSQ, PAY), jnp.bfloat16),
            pltpu.VMEM((LOG2, BSQ, PAY), jnp.bfloat16),
            pltpu.SemaphoreType.DMA((LOG2,)),
            pltpu.SemaphoreType.DMA((LOG2,)),
        ],
        compiler_params=pltpu.CompilerParams(collective_id=0),
    )(x, Wq, Wo, K_ext, V_ext)


# device time: 48136 ns/iter; 1.6630x vs baseline; 1.0018x over previous
import jax
import jax.numpy as jnp
from jax import lax
from jax.experimental import pallas as pl
from jax.experimental.pallas import tpu as pltpu

N_DEV = 32
LOG2 = 5
B, SQ, D, HQ, DH = 2, 128, 512, 8, 64
BSQ = B * SQ
PAY = 640


def kernel(x, Wq, Wo, K_ext, V_ext):
    skv = K_ext.shape[1]

    def body(x_ref, wq_ref, wo_ref, k_ref, v_ref, out_ref,
             acc_ref, send_ref, recv_ref, send_sems, recv_sems):
        my_i = lax.axis_index("i")

        xb = x_ref[...].reshape(BSQ, D).astype(jnp.bfloat16)
        q2d = jax.lax.dot(xb, wq_ref[...].astype(jnp.bfloat16),
                          preferred_element_type=jnp.float32)
        q2d = q2d.astype(jnp.bfloat16)

        acc_ref[:, pl.ds(512, 128)] = jnp.zeros((BSQ, 128), jnp.float32)
        for b in range(B):
            kb = k_ref[b].reshape(skv, HQ * DH).astype(jnp.bfloat16)
            vb = v_ref[b].reshape(skv, HQ * DH).astype(jnp.bfloat16)
            lsums = []
            for h in range(HQ):
                qbh = q2d[b * SQ:(b + 1) * SQ, h * DH:(h + 1) * DH]
                kbh = kb[:, h * DH:(h + 1) * DH]
                s = jax.lax.dot_general(
                    qbh, kbh, (((1,), (1,)), ((), ())),
                    preferred_element_type=jnp.float32) * 0.125
                p = jnp.exp(s)
                lsums.append(jnp.sum(p, axis=1, keepdims=True))
                vbh = vb[:, h * DH:(h + 1) * DH]
                o = jax.lax.dot(p.astype(jnp.bfloat16), vbh,
                                preferred_element_type=jnp.float32)
                acc_ref[pl.ds(b * SQ, SQ), pl.ds(h * DH, DH)] = o
            acc_ref[pl.ds(b * SQ, SQ), pl.ds(512, HQ)] = jnp.concatenate(
                lsums, axis=1)

        barrier = pltpu.get_barrier_semaphore()
        for s in range(LOG2):
            partner = my_i ^ (1 << s)
            pl.semaphore_signal(barrier, inc=1, device_id=(partner,),
                                device_id_type=pl.DeviceIdType.MESH)
        pl.semaphore_wait(barrier, LOG2)

        for s in range(LOG2):
            partner = my_i ^ (1 << s)
            send_ref[...] = acc_ref[...].astype(jnp.bfloat16)
            rdma = pltpu.make_async_remote_copy(
                src_ref=send_ref,
                dst_ref=recv_ref.at[s],
                send_sem=send_sems.at[s],
                recv_sem=recv_sems.at[s],
                device_id=(partner,),
                device_id_type=pl.DeviceIdType.MESH,
            )
            rdma.start()
            rdma.wait()
            acc_ref[...] += recv_ref[s].astype(jnp.float32)

        accv = acc_ref[...]
        for h in range(HQ):
            acc_ref[:, pl.ds(h * DH, DH)] = (
                accv[:, h * DH:(h + 1) * DH] / accv[:, 512 + h:513 + h]
            )
        onorm = acc_ref[:, pl.ds(0, 512)].astype(jnp.bfloat16)
        res = jax.lax.dot(onorm, wo_ref[...].astype(jnp.bfloat16),
                          preferred_element_type=jnp.float32)
        out_ref[...] = res.reshape(B, SQ, D)

    return pl.pallas_call(
        body,
        out_shape=jax.ShapeDtypeStruct((B, SQ, D), jnp.float32),
        in_specs=[pl.BlockSpec(memory_space=pltpu.VMEM)] * 5,
        out_specs=pl.BlockSpec(memory_space=pltpu.VMEM),
        scratch_shapes=[
            pltpu.VMEM((BSQ, PAY), jnp.float32),
            pltpu.VMEM((BSQ, PAY), jnp.bfloat16),
            pltpu.VMEM((LOG2, BSQ, PAY), jnp.bfloat16),
            pltpu.SemaphoreType.DMA((LOG2,)),
            pltpu.SemaphoreType.DMA((LOG2,)),
        ],
        compiler_params=pltpu.CompilerParams(collective_id=0),
    )(x, Wq, Wo, K_ext, V_ext)


# device time: 39933 ns/iter; 2.0046x vs baseline; 1.2054x over previous
import jax
import jax.numpy as jnp
from jax import lax
from jax.experimental import pallas as pl
from jax.experimental.pallas import tpu as pltpu

N_DEV = 32
LOG2 = 5
B, SQ, D, HQ, DH = 2, 128, 512, 8, 64
BSQ = B * SQ
PAY = 640


def kernel(x, Wq, Wo, K_ext, V_ext):
    skv = K_ext.shape[1]

    def body(x_ref, wq_ref, wo_ref, k_ref, v_ref, out_ref,
             acc_ref, recv_ref, send_sems, recv_sems):
        my_i = lax.axis_index("i")

        barrier = pltpu.get_barrier_semaphore()
        for s in range(LOG2):
            pl.semaphore_signal(barrier, inc=1,
                                device_id=(my_i ^ (1 << s),),
                                device_id_type=pl.DeviceIdType.MESH)
        pl.semaphore_wait(barrier, LOG2)

        xb = x_ref[...].reshape(BSQ, D).astype(jnp.bfloat16)
        q2d = jax.lax.dot(xb, wq_ref[...].astype(jnp.bfloat16),
                          preferred_element_type=jnp.float32)
        q2d = q2d.astype(jnp.bfloat16)

        def compute_partial(b):
            kb = k_ref[b].reshape(skv, HQ * DH).astype(jnp.bfloat16)
            vb = v_ref[b].reshape(skv, HQ * DH).astype(jnp.bfloat16)
            lsums = []
            for h in range(HQ):
                qbh = q2d[b * SQ:(b + 1) * SQ, h * DH:(h + 1) * DH]
                s = jax.lax.dot_general(
                    qbh, kb[:, h * DH:(h + 1) * DH], (((1,), (1,)), ((), ())),
                    preferred_element_type=jnp.float32) * 0.125
                p = jnp.exp(s)
                lsums.append(jnp.sum(p, axis=1, keepdims=True))
                o = jax.lax.dot(p.astype(jnp.bfloat16),
                                vb[:, h * DH:(h + 1) * DH],
                                preferred_element_type=jnp.float32)
                acc_ref[pl.ds(b * SQ, SQ), pl.ds(h * DH, DH)] = (
                    o.astype(jnp.bfloat16))
            acc_ref[pl.ds(b * SQ, SQ), pl.ds(512, 128)] = jnp.concatenate(
                lsums + [jnp.zeros((SQ, 128 - HQ), jnp.float32)], axis=1
            ).astype(jnp.bfloat16)

        def mk(st, s):
            return pltpu.make_async_remote_copy(
                src_ref=acc_ref.at[pl.ds(st * SQ, SQ)],
                dst_ref=recv_ref.at[st, s],
                send_sem=send_sems.at[st, s],
                recv_sem=recv_sems.at[st, s],
                device_id=(my_i ^ (1 << s),),
                device_id_type=pl.DeviceIdType.MESH,
            )

        compute_partial(0)
        mk(0, 0).start()
        compute_partial(1)
        mk(1, 0).start()
        for s in range(LOG2):
            for st in range(2):
                mk(st, s).wait()
                rows = pl.ds(st * SQ, SQ)
                acc_ref[rows, :] = acc_ref[rows, :] + recv_ref[st, s]
                if s < LOG2 - 1:
                    mk(st, s + 1).start()

        accv = acc_ref[...].astype(jnp.float32)
        parts = []
        for h in range(HQ):
            parts.append(accv[:, h * DH:(h + 1) * DH]
                         / accv[:, 512 + h:513 + h])
        onorm = jnp.concatenate(parts, axis=1).astype(jnp.bfloat16)
        res = jax.lax.dot(onorm, wo_ref[...].astype(jnp.bfloat16),
                          preferred_element_type=jnp.float32)
        out_ref[...] = res.reshape(B, SQ, D)

    return pl.pallas_call(
        body,
        out_shape=jax.ShapeDtypeStruct((B, SQ, D), jnp.float32),
        in_specs=[pl.BlockSpec(memory_space=pltpu.VMEM)] * 5,
        out_specs=pl.BlockSpec(memory_space=pltpu.VMEM),
        scratch_shapes=[
            pltpu.VMEM((BSQ, PAY), jnp.bfloat16),
            pltpu.VMEM((2, LOG2, SQ, PAY), jnp.bfloat16),
            pltpu.SemaphoreType.DMA((2, LOG2)),
            pltpu.SemaphoreType.DMA((2, LOG2)),
        ],
        compiler_params=pltpu.CompilerParams(collective_id=0),
    )(x, Wq, Wo, K_ext, V_ext)
